# baseline (speedup 1.0000x reference)
import functools

import jax
import jax.numpy as jnp
from jax.experimental import pallas as pl

_N_NBR = 160000
_N_NODES = 10000
_F = 128
_H = 256
_E = 256

_RBLK = 2000
_DBLK = 2000


def _reduce_body(l1_ref, l2_ref, s1_ref, s2_ref):
    i = pl.program_id(0)

    @pl.when(i == 0)
    def _():
        s1_ref[...] = jnp.zeros_like(s1_ref)
        s2_ref[...] = jnp.zeros_like(s2_ref)

    s1_ref[...] += jnp.sum(l1_ref[...], axis=0, keepdims=True)
    s2_ref[...] += jnp.sum(l2_ref[...], axis=0, keepdims=True)


def _ln_relu(x, g, b, eps=1e-5):
    mu = jnp.mean(x, axis=-1, keepdims=True)
    var = jnp.mean((x - mu) ** 2, axis=-1, keepdims=True)
    y = (x - mu) / jnp.sqrt(var + eps) * g + b
    return jnp.maximum(y, 0.0)


def _dense_body(nf_ref, s1_ref, s2_ref, ws1_ref, wn1_ref, c1b_ref, g1_ref,
                be1_ref, ws2_ref, wn2_ref, c2b_ref, g2_ref, be2_ref,
                h2_ref, ssq_ref):
    i = pl.program_id(0)
    inv = 1.0 / _N_NBR
    agg1 = s1_ref[...] * inv
    agg2 = s2_ref[...] * inv
    c1 = jnp.dot(agg1, wn1_ref[...], preferred_element_type=jnp.float32) + c1b_ref[...]
    out1 = jnp.dot(nf_ref[...], ws1_ref[...], preferred_element_type=jnp.float32) + c1
    h1 = _ln_relu(out1, g1_ref[...], be1_ref[...])
    c2 = jnp.dot(agg2, wn2_ref[...], preferred_element_type=jnp.float32) + c2b_ref[...]
    out2 = jnp.dot(h1, ws2_ref[...], preferred_element_type=jnp.float32) + c2
    h2 = _ln_relu(out2, g2_ref[...], be2_ref[...])
    h2_ref[...] = h2

    @pl.when(i == 0)
    def _():
        ssq_ref[...] = jnp.zeros_like(ssq_ref)

    ssq_ref[...] += jnp.sum(h2 * h2, axis=0, keepdims=True)


def _norm_body(h2_ref, ssq_ref, out_ref):
    norm = jnp.sqrt(ssq_ref[...])
    out_ref[...] = h2_ref[...] / jnp.maximum(norm, 1e-12)


@jax.jit
def kernel(node_feat, neighbor_feats_l1, neighbor_feats_l2, W_self1, b_self1,
           W_nbr1, b_nbr1, g1, be1, W_self2, b_self2, W_nbr2, b_nbr2, g2, be2):
    f32 = jnp.float32

    sum1, sum2 = pl.pallas_call(
        _reduce_body,
        grid=(_N_NBR // _RBLK,),
        in_specs=[
            pl.BlockSpec((_RBLK, _F), lambda i: (i, 0)),
            pl.BlockSpec((_RBLK, _H), lambda i: (i, 0)),
        ],
        out_specs=[
            pl.BlockSpec((1, _F), lambda i: (0, 0)),
            pl.BlockSpec((1, _H), lambda i: (0, 0)),
        ],
        out_shape=[
            jax.ShapeDtypeStruct((1, _F), f32),
            jax.ShapeDtypeStruct((1, _H), f32),
        ],
    )(neighbor_feats_l1, neighbor_feats_l2)

    c1b = (b_self1 + b_nbr1).reshape(1, _H)
    c2b = (b_self2 + b_nbr2).reshape(1, _E)

    h2, ssq = pl.pallas_call(
        _dense_body,
        grid=(_N_NODES // _DBLK,),
        in_specs=[
            pl.BlockSpec((_DBLK, _F), lambda i: (i, 0)),
            pl.BlockSpec((1, _F), lambda i: (0, 0)),
            pl.BlockSpec((1, _H), lambda i: (0, 0)),
            pl.BlockSpec((_F, _H), lambda i: (0, 0)),
            pl.BlockSpec((_F, _H), lambda i: (0, 0)),
            pl.BlockSpec((1, _H), lambda i: (0, 0)),
            pl.BlockSpec((1, _H), lambda i: (0, 0)),
            pl.BlockSpec((1, _H), lambda i: (0, 0)),
            pl.BlockSpec((_H, _E), lambda i: (0, 0)),
            pl.BlockSpec((_H, _E), lambda i: (0, 0)),
            pl.BlockSpec((1, _E), lambda i: (0, 0)),
            pl.BlockSpec((1, _E), lambda i: (0, 0)),
            pl.BlockSpec((1, _E), lambda i: (0, 0)),
        ],
        out_specs=[
            pl.BlockSpec((_DBLK, _E), lambda i: (i, 0)),
            pl.BlockSpec((1, _E), lambda i: (0, 0)),
        ],
        out_shape=[
            jax.ShapeDtypeStruct((_N_NODES, _E), f32),
            jax.ShapeDtypeStruct((1, _E), f32),
        ],
    )(node_feat, sum1, sum2, W_self1, W_nbr1, c1b, g1.reshape(1, _H),
      be1.reshape(1, _H), W_self2, W_nbr2, c2b, g2.reshape(1, _E),
      be2.reshape(1, _E))

    out = pl.pallas_call(
        _norm_body,
        grid=(_N_NODES // _DBLK,),
        in_specs=[
            pl.BlockSpec((_DBLK, _E), lambda i: (i, 0)),
            pl.BlockSpec((1, _E), lambda i: (0, 0)),
        ],
        out_specs=pl.BlockSpec((_DBLK, _E), lambda i: (i, 0)),
        out_shape=jax.ShapeDtypeStruct((_N_NODES, _E), f32),
    )(h2, ssq)

    return out
